# baseline (speedup 1.0000x reference)
import functools

import jax
import jax.numpy as jnp
from jax import lax
from jax.experimental import pallas as pl
from jax.experimental.pallas import tpu as pltpu
from jax.experimental.pallas import tpu_sc as plsc

NC = 2
NS = 16
NW = NC * NS
L = 16

D = 64
CHUNK = 64


@functools.lru_cache(maxsize=None)
def _make_sc_scorer(total):
    assert total % (NW * CHUNK) == 0
    per_w = total // NW
    n_chunks = per_w // CHUNK
    mesh = plsc.VectorSubcoreMesh(core_axis_name="c", subcore_axis_name="s")

    @functools.partial(
        pl.kernel,
        mesh=mesh,
        out_type=jax.ShapeDtypeStruct((total,), jnp.float32),
        compiler_params=pltpu.CompilerParams(needs_layout_passes=False),
        scratch_types=[
            pltpu.VMEM((CHUNK,), jnp.int32),
            pltpu.VMEM((CHUNK,), jnp.int32),
            pltpu.VMEM((CHUNK,), jnp.int32),
            pltpu.VMEM((CHUNK, D), jnp.float32),
            pltpu.VMEM((CHUNK, D), jnp.float32),
            pltpu.VMEM((CHUNK, D), jnp.float32),
            pltpu.VMEM((CHUNK, D), jnp.float32),
            pltpu.VMEM((CHUNK, D), jnp.float32),
            pltpu.VMEM((CHUNK, D), jnp.float32),
            pltpu.VMEM((per_w,), jnp.float32),
            pltpu.SemaphoreType.DMA,
        ],
    )
    def scorer(h_hbm, t_hbm, r_hbm, eh_hbm, et_hbm, relh_hbm, relt_hbm,
               out_hbm, hs, ts, rs, hh, ht, th, tt, rh, rt, sv, sem):
        wid = lax.axis_index("s") * NC + lax.axis_index("c")
        base = wid * per_w
        lanes = lax.iota(jnp.int32, L)

        def chunk_body(c, carry):
            off = base + c * CHUNK
            pltpu.sync_copy(h_hbm.at[pl.ds(off, CHUNK)], hs)
            pltpu.sync_copy(t_hbm.at[pl.ds(off, CHUNK)], ts)
            pltpu.sync_copy(r_hbm.at[pl.ds(off, CHUNK)], rs)

            copies = []
            for g in range(CHUNK // L):
                i0 = g * L
                hvec = hs[pl.ds(i0, L)]
                tvec = ts[pl.ds(i0, L)]
                rvec = rs[pl.ds(i0, L)]
                for lane in range(L):
                    j = i0 + lane
                    h = hvec[lane] * D
                    t = tvec[lane] * D
                    r = rvec[lane] * D
                    copies += [
                        pltpu.async_copy(eh_hbm.at[pl.ds(h, D)], hh.at[j],
                                         sem),
                        pltpu.async_copy(et_hbm.at[pl.ds(h, D)], ht.at[j],
                                         sem),
                        pltpu.async_copy(eh_hbm.at[pl.ds(t, D)], th.at[j],
                                         sem),
                        pltpu.async_copy(et_hbm.at[pl.ds(t, D)], tt.at[j],
                                         sem),
                        pltpu.async_copy(relh_hbm.at[pl.ds(r, D)], rh.at[j],
                                         sem),
                        pltpu.async_copy(relt_hbm.at[pl.ds(r, D)], rt.at[j],
                                         sem),
                    ]
            for cp in copies:
                cp.wait()

            def group_body(g, carry2):
                i0 = g * L
                rows = lanes + i0
                acc = jnp.zeros((L,), jnp.float32)
                for d in range(D):
                    dvec = jnp.full((L,), d, jnp.int32)
                    hhd = plsc.load_gather(hh, [rows, dvec])
                    htd = plsc.load_gather(ht, [rows, dvec])
                    thd = plsc.load_gather(th, [rows, dvec])
                    ttd = plsc.load_gather(tt, [rows, dvec])
                    rhd = plsc.load_gather(rh, [rows, dvec])
                    rtd = plsc.load_gather(rt, [rows, dvec])
                    acc = acc + (hhd * rhd * ttd + thd * rtd * htd)
                sv[pl.ds(c * CHUNK + i0, L)] = 0.5 * acc
                return carry2

            lax.fori_loop(0, CHUNK // L, group_body, 0)
            return carry

        lax.fori_loop(0, n_chunks, chunk_body, 0)
        pltpu.sync_copy(sv, out_hbm.at[pl.ds(base, per_w)])

    return scorer


def kernel(pos_h, pos_r, pos_t, neg_h, neg_r, neg_t,
           entity_head, entity_tail, relation_head, relation_tail):
    b = pos_h.shape[0]
    h = jnp.concatenate([pos_h, neg_h])
    t = jnp.concatenate([pos_t, neg_t])
    r = jnp.concatenate([pos_r, neg_r])
    scorer = _make_sc_scorer(2 * b)
    out = scorer(h, t, r, entity_head.reshape(-1), entity_tail.reshape(-1),
                 relation_head.reshape(-1), relation_tail.reshape(-1))
    return out[:b], out[b:]
